# baseline (speedup 1.0000x reference)
import functools

import jax
import jax.numpy as jnp
from jax import lax
from jax.experimental import pallas as pl
from jax.experimental.pallas import tpu as pltpu
from jax.experimental.pallas import tpu_sc as plsc

_L = 16


@functools.cache
def _build(B, HL, D, NE):
    info = plsc.get_sparse_core_info()
    NC, NS = info.num_cores, info.num_subcores
    NW = NC * NS
    D2 = 2 * D
    assert B % NW == 0
    bags_per_w = B // NW
    idx_per_w = bags_per_w * HL
    CB = 32
    n_chunks = bags_per_w // CB
    idx_per_chunk = CB * HL
    n_vec = idx_per_chunk // _L

    mesh = plsc.VectorSubcoreMesh(core_axis_name="c", subcore_axis_name="s")

    @functools.partial(
        pl.kernel,
        out_type=(
            jax.ShapeDtypeStruct((B, D2), jnp.float32),
            jax.ShapeDtypeStruct((B * HL,), jnp.int32),
        ),
        mesh=mesh,
        scratch_types=[
            pltpu.VMEM((idx_per_chunk,), jnp.int32),
            pltpu.VMEM((idx_per_chunk + _L,), jnp.int32),
            pltpu.VMEM((idx_per_chunk,), jnp.int32),
            pltpu.VMEM((idx_per_chunk, D2), jnp.float32),
            pltpu.VMEM((CB, D2), jnp.float32),
            pltpu.SemaphoreType.DMA,
        ],
    )
    def k(vals_hbm, table_hbm, out_hbm, remap_hbm, vals_v, idx_v, pidx_v, rows_v,
          out_v, sem):
        wid = lax.axis_index("s") * NC + lax.axis_index("c")
        base = wid * idx_per_w

        def chunk_body(c, _):
            off = base + c * idx_per_chunk
            pltpu.sync_copy(vals_hbm.at[pl.ds(off, idx_per_chunk)], vals_v)

            def remap_body(j, _):
                v = vals_v[pl.ds(j * _L, _L)]
                r = (v * 31 + 17) % NE
                idx_v[pl.ds(j * _L, _L)] = r
                pidx_v[pl.ds(j * _L, _L)] = r >> 1
                return 0

            lax.fori_loop(0, n_vec, remap_body, 0)
            pltpu.sync_copy(idx_v.at[pl.ds(0, idx_per_chunk)],
                            remap_hbm.at[pl.ds(off, idx_per_chunk)])
            pltpu.async_copy(table_hbm.at[pidx_v], rows_v, sem).wait()

            def bag_body(b, _):
                r0 = b * HL
                hv0 = (idx_v[pl.ds(r0, _L)] & 1) * D
                hv1 = (idx_v[pl.ds(r0 + _L, _L)] & 1) * D
                accs = [jnp.zeros((_L,), jnp.float32) for _ in range(D // _L)]
                for l in range(HL):
                    half = hv0[l] if l < _L else hv1[l - _L]
                    for d in range(D // _L):
                        accs[d] += rows_v[r0 + l, pl.ds(half + d * _L, _L)]
                for d in range(D // _L):
                    out_v[b, pl.ds(d * _L, _L)] = accs[d]
                return 0

            lax.fori_loop(0, CB, bag_body, 0)
            pltpu.sync_copy(out_v, out_hbm.at[pl.ds(wid * bags_per_w + c * CB, CB)])
            return 0

        lax.fori_loop(0, n_chunks, chunk_body, 0)

    return k


def kernel(values, table):
    B, HL = values.shape
    NE, D = table.shape
    table2 = table.reshape(NE // 2, 2 * D)
    pooled, remap = _build(B, HL, D, NE)(values.reshape(-1), table2)
    return pooled[:, :D], remap.reshape(B, HL)


# speedup vs baseline: 1.1025x; 1.1025x over previous
import functools

import jax
import jax.numpy as jnp
from jax import lax
from jax.experimental import pallas as pl
from jax.experimental.pallas import tpu as pltpu
from jax.experimental.pallas import tpu_sc as plsc

_L = 16


@functools.cache
def _build(B, HL, D, NE):
    info = plsc.get_sparse_core_info()
    NC, NS = info.num_cores, info.num_subcores
    NW = NC * NS
    D2 = 2 * D
    assert B % NW == 0
    bags_per_w = B // NW
    idx_per_w = bags_per_w * HL
    CB = 32
    n_chunks = bags_per_w // CB
    idx_per_chunk = CB * HL
    n_vec = idx_per_chunk // _L

    mesh = plsc.VectorSubcoreMesh(core_axis_name="c", subcore_axis_name="s")

    @functools.partial(
        pl.kernel,
        out_type=(
            jax.ShapeDtypeStruct((B, D2), jnp.float32),
            jax.ShapeDtypeStruct((B * HL,), jnp.int32),
        ),
        mesh=mesh,
        scratch_types=[
            pltpu.VMEM((idx_per_chunk,), jnp.int32),
            pltpu.VMEM((idx_per_chunk,), jnp.int32),
            pltpu.VMEM((idx_per_chunk, D2), jnp.float32),
            pltpu.VMEM((CB, D2), jnp.float32),
            pltpu.SemaphoreType.DMA,
        ],
    )
    def k(vals_hbm, table_hbm, out_hbm, remap_hbm, vals_v, idx_v, rows_v,
          out_v, sem):
        wid = lax.axis_index("s") * NC + lax.axis_index("c")
        base = wid * idx_per_w

        def chunk_body(c, _):
            off = base + c * idx_per_chunk
            pltpu.sync_copy(vals_hbm.at[pl.ds(off, idx_per_chunk)], vals_v)

            def remap_body(j, _):
                v = vals_v[pl.ds(j * _L, _L)]
                idx_v[pl.ds(j * _L, _L)] = (v * 31 + 17) % NE
                return 0

            lax.fori_loop(0, n_vec, remap_body, 0)
            pltpu.sync_copy(idx_v, remap_hbm.at[pl.ds(off, idx_per_chunk)])
            pltpu.async_copy(table_hbm.at[idx_v], rows_v, sem).wait()

            def bag_body(b, _):
                r0 = b * HL
                accs = [rows_v[r0, pl.ds(d * _L, _L)] for d in range(D // _L)]
                for l in range(1, HL):
                    for d in range(D // _L):
                        accs[d] += rows_v[r0 + l, pl.ds(d * _L, _L)]
                for d in range(D // _L):
                    out_v[b, pl.ds(d * _L, _L)] = accs[d]
                return 0

            lax.fori_loop(0, CB, bag_body, 0)
            pltpu.sync_copy(out_v, out_hbm.at[pl.ds(wid * bags_per_w + c * CB, CB)])
            return 0

        lax.fori_loop(0, n_chunks, chunk_body, 0)

    return k


def kernel(values, table):
    B, HL = values.shape
    NE, D = table.shape
    table2 = jnp.pad(table, ((0, 0), (0, D)))
    pooled, remap = _build(B, HL, D, NE)(values.reshape(-1), table2)
    return pooled[:, :D], remap.reshape(B, HL)


# speedup vs baseline: 1.1518x; 1.0447x over previous
import functools

import jax
import jax.numpy as jnp
from jax import lax
from jax.experimental import pallas as pl
from jax.experimental.pallas import tpu as pltpu
from jax.experimental.pallas import tpu_sc as plsc

_L = 16


@functools.cache
def _build(B, HL, D, NE):
    info = plsc.get_sparse_core_info()
    NC, NS = info.num_cores, info.num_subcores
    NW = NC * NS
    D2 = 2 * D
    assert B % NW == 0
    bags_per_w = B // NW
    idx_per_w = bags_per_w * HL
    CB = 16
    n_chunks = bags_per_w // CB
    n_pairs = n_chunks // 2
    ipc = CB * HL
    n_vec = idx_per_w // _L

    mesh = plsc.VectorSubcoreMesh(core_axis_name="c", subcore_axis_name="s")

    @functools.partial(
        pl.kernel,
        out_type=(
            jax.ShapeDtypeStruct((B, D2), jnp.float32),
            jax.ShapeDtypeStruct((B * HL,), jnp.int32),
        ),
        mesh=mesh,
        scratch_types=[
            pltpu.VMEM((idx_per_w,), jnp.int32),
            pltpu.VMEM((idx_per_w,), jnp.int32),
            pltpu.VMEM((ipc, D2), jnp.float32),
            pltpu.VMEM((ipc, D2), jnp.float32),
            pltpu.VMEM((CB, D2), jnp.float32),
            pltpu.VMEM((CB, D2), jnp.float32),
            pltpu.SemaphoreType.DMA,
            pltpu.SemaphoreType.DMA,
            pltpu.SemaphoreType.DMA,
            pltpu.SemaphoreType.DMA,
        ],
    )
    def k(vals_hbm, table_hbm, out_hbm, remap_hbm, vals_v, idx_v,
          rows_a, rows_b, out_a, out_b, gsem_a, gsem_b, osem_a, osem_b):
        wid = lax.axis_index("s") * NC + lax.axis_index("c")
        base = wid * idx_per_w
        obase = wid * bags_per_w
        pltpu.sync_copy(vals_hbm.at[pl.ds(base, idx_per_w)], vals_v)

        def remap_body(j, _):
            v = vals_v[pl.ds(j * _L, _L)]
            idx_v[pl.ds(j * _L, _L)] = (v * 31 + 17) % NE
            return 0

        lax.fori_loop(0, n_vec, remap_body, 0)
        pltpu.sync_copy(idx_v, remap_hbm.at[pl.ds(base, idx_per_w)])

        def gather(c, rows, sem):
            return pltpu.async_copy(
                table_hbm.at[idx_v.at[pl.ds(c * ipc, ipc)]], rows, sem)

        def gather_wait(c, rows, sem):
            pltpu.make_async_copy(
                table_hbm.at[idx_v.at[pl.ds(c * ipc, ipc)]], rows, sem).wait()

        def pool(c, rows, out_v, osem):
            def bag_body(b, _):
                r0 = b * HL
                accs = [rows[r0, pl.ds(d * _L, _L)] for d in range(D // _L)]
                for l in range(1, HL):
                    for d in range(D // _L):
                        accs[d] += rows[r0 + l, pl.ds(d * _L, _L)]
                for d in range(D // _L):
                    out_v[b, pl.ds(d * _L, _L)] = accs[d]
                return 0

            lax.fori_loop(0, CB, bag_body, 0)
            pltpu.async_copy(out_v, out_hbm.at[pl.ds(obase + c * CB, CB)], osem)

        def out_wait(out_v, osem):
            pltpu.make_async_copy(out_v, out_hbm.at[pl.ds(obase, CB)], osem).wait()

        gather(0, rows_a, gsem_a)

        def pair_body(c2, _):
            c0 = 2 * c2
            gather(c0 + 1, rows_b, gsem_b)
            gather_wait(c0, rows_a, gsem_a)

            @pl.when(c2 > 0)
            def _():
                out_wait(out_a, osem_a)

            pool(c0, rows_a, out_a, osem_a)

            @pl.when(c2 < n_pairs - 1)
            def _():
                gather(c0 + 2, rows_a, gsem_a)

            gather_wait(c0 + 1, rows_b, gsem_b)

            @pl.when(c2 > 0)
            def _():
                out_wait(out_b, osem_b)

            pool(c0 + 1, rows_b, out_b, osem_b)
            return 0

        lax.fori_loop(0, n_pairs, pair_body, 0)
        out_wait(out_a, osem_a)
        out_wait(out_b, osem_b)

    return k


def kernel(values, table):
    B, HL = values.shape
    NE, D = table.shape
    table2 = jnp.pad(table, ((0, 0), (0, D)))
    pooled, remap = _build(B, HL, D, NE)(values.reshape(-1), table2)
    return pooled[:, :D], remap.reshape(B, HL)
